# baseline (speedup 1.0000x reference)
import functools

import jax
import jax.numpy as jnp
from jax import lax
from jax.experimental import pallas as pl
from jax.experimental.pallas import tpu as pltpu
from jax.experimental.pallas import tpu_sc as plsc

NUM_EMBEDDINGS = 100000
EMBEDDING_DIM = 128
OUT_DIM = 32

ROW_BLOCK = 4000


def _proj_body(table_ref, w_ref, b_ref, out_ref):
    t4 = table_ref[...].reshape(ROW_BLOCK // 4, 4, EMBEDDING_DIM)
    for k in range(4):
        acc = lax.dot_general(
            t4[:, k, :], w_ref[...],
            dimension_numbers=(((1,), (1,)), ((), ())),
            preferred_element_type=jnp.float32,
        )
        out_ref[:, k * OUT_DIM:(k + 1) * OUT_DIM] = acc + b_ref[...]


def _project_table(table, W, b2d):
    grid = NUM_EMBEDDINGS // ROW_BLOCK
    return pl.pallas_call(
        _proj_body,
        grid=(grid,),
        in_specs=[
            pl.BlockSpec((ROW_BLOCK, EMBEDDING_DIM), lambda i: (i, 0)),
            pl.BlockSpec((OUT_DIM, EMBEDDING_DIM), lambda i: (0, 0)),
            pl.BlockSpec((1, OUT_DIM), lambda i: (0, 0)),
        ],
        out_specs=pl.BlockSpec((ROW_BLOCK // 4, 4 * OUT_DIM), lambda i: (i, 0)),
        out_shape=jax.ShapeDtypeStruct((NUM_EMBEDDINGS // 4, 4 * OUT_DIM), jnp.float32),
    )(table, W, b2d)


_INFO = plsc.get_sparse_core_info()
_NC = _INFO.num_cores
_NS = _INFO.num_subcores
_NW = _NC * _NS

_B = 4096
_L = 50
_NB = _B // _NW
_NBUF = 2


def _make_lookup():
    mesh = plsc.VectorSubcoreMesh(core_axis_name="c", subcore_axis_name="s")

    @functools.partial(
        pl.kernel,
        mesh=mesh,
        out_type=jax.ShapeDtypeStruct(
            (_L, OUT_DIM // 8, _B // 128, 8, 128), jnp.float32),
        scratch_types=(
            [pltpu.VMEM((_NB * _L,), jnp.int32)]
            + [pltpu.VMEM((_NB,), jnp.int32) for _ in range(_NBUF)]
            + [pltpu.VMEM((_NB, OUT_DIM), jnp.float32) for _ in range(_NBUF)]
            + [pltpu.VMEM((OUT_DIM, 129), jnp.float32) for _ in range(_NBUF)]
            + [pltpu.SemaphoreType.DMA for _ in range(2 * _NBUF)]
        ),
        compiler_params=pltpu.CompilerParams(
            use_tc_tiling_on_sc=False, needs_layout_passes=False
        ),
    )
    def lookup_k(idx_hbm, proj_hbm, out_hbm, idx_v, *bufs):
        cols = list(bufs[0:_NBUF])
        rows = list(bufs[_NBUF:2 * _NBUF])
        tiles = list(bufs[2 * _NBUF:3 * _NBUF])
        gsems = list(bufs[3 * _NBUF:4 * _NBUF])
        wsems = list(bufs[4 * _NBUF:5 * _NBUF])
        wid = lax.axis_index("s") * _NC + lax.axis_index("c")
        iota16 = lax.iota(jnp.int32, 16)
        iota50 = iota16 * _L

        pltpu.sync_copy(idx_hbm.at[pl.ds(wid * (_NB * _L), _NB * _L)], idx_v)

        def build_col(l, col):
            for k in range(8):
                vals = plsc.load_gather(idx_v, [iota50 + (16 * _L * k + l)])
                col[pl.ds(16 * k, 16)] = vals

        iota16hi = iota16 + 16

        def transpose(rows_v, tile_v):
            for g in range(_NB // 8):
                vals = []
                for j in range(8):
                    bi = g * 8 + j
                    vals.append((bi, rows_v[bi, pl.ds(0, 16)],
                                 rows_v[bi, pl.ds(16, 16)]))
                for bi, v0, v1 in vals:
                    bvec = jnp.zeros((16,), jnp.int32) + bi
                    plsc.store_scatter(tile_v, [iota16, bvec], v0)
                    plsc.store_scatter(tile_v, [iota16hi, bvec], v1)

        for l in range(_NBUF - 1):
            build_col(l, cols[l])
            pltpu.async_copy(proj_hbm.at[cols[l]], rows[l], gsems[l])

        def block(i, carry):
            for p in range(_NBUF):
                l = i * _NBUF + p
                nxt = (p + _NBUF - 1) % _NBUF

                @pl.when(l + _NBUF - 1 < _L)
                def _():
                    build_col(l + _NBUF - 1, cols[nxt])
                    pltpu.async_copy(proj_hbm.at[cols[nxt]], rows[nxt], gsems[nxt])

                pltpu.make_async_copy(
                    proj_hbm.at[cols[p]], rows[p], gsems[p]
                ).wait()

                @pl.when(i > 0)
                def _():
                    pltpu.make_async_copy(
                        rows[p], proj_hbm.at[pl.ds(0, _NB)], wsems[p]
                    ).wait()

                transpose(rows[p], tiles[p])
                for ot in range(4):
                    pltpu.async_copy(
                        tiles[p].at[pl.ds(ot * 8, 8), pl.ds(0, 128)],
                        out_hbm.at[l, ot, wid],
                        wsems[p],
                    )
            return carry

        lax.fori_loop(0, _L // _NBUF, block, 0, unroll=False)
        for p in range(_NBUF):
            pltpu.make_async_copy(
                rows[p], proj_hbm.at[pl.ds(0, _NB)], wsems[p]
            ).wait()

    return lookup_k


def kernel(x, table, W, b):
    proj = _project_table(table, W, b.reshape(1, OUT_DIM))
    proj = proj.reshape(NUM_EMBEDDINGS, OUT_DIM)
    out6 = _make_lookup()(x.reshape(_B * _L), proj)
    return out6.transpose(2, 4, 0, 1, 3).reshape(_B, _L, OUT_DIM)


# speedup vs baseline: 1.0419x; 1.0419x over previous
import functools

import jax
import jax.numpy as jnp
from jax import lax
from jax.experimental import pallas as pl
from jax.experimental.pallas import tpu as pltpu
from jax.experimental.pallas import tpu_sc as plsc

NUM_EMBEDDINGS = 100000
EMBEDDING_DIM = 128
OUT_DIM = 32

_QUARTER = NUM_EMBEDDINGS // 4
ROW_BLOCK = 1000


def _proj_body(t0, t1, t2, t3, w_ref, b_ref, out_ref):
    for a, t in enumerate((t0, t1, t2, t3)):
        acc = lax.dot_general(
            t[...], w_ref[...],
            dimension_numbers=(((1,), (1,)), ((), ())),
            preferred_element_type=jnp.float32,
        )
        out_ref[:, a * OUT_DIM:(a + 1) * OUT_DIM] = acc + b_ref[...]


def _project_table(table, W, b2d):
    grid = _QUARTER // ROW_BLOCK
    tspec = lambda a: pl.BlockSpec(
        (ROW_BLOCK, EMBEDDING_DIM), lambda i, a=a: (i + grid * a, 0))
    return pl.pallas_call(
        _proj_body,
        grid=(grid,),
        in_specs=[
            tspec(0), tspec(1), tspec(2), tspec(3),
            pl.BlockSpec((OUT_DIM, EMBEDDING_DIM), lambda i: (0, 0)),
            pl.BlockSpec((1, OUT_DIM), lambda i: (0, 0)),
        ],
        out_specs=pl.BlockSpec((ROW_BLOCK, 4 * OUT_DIM), lambda i: (i, 0)),
        out_shape=jax.ShapeDtypeStruct((_QUARTER, 4 * OUT_DIM), jnp.float32),
    )(table, table, table, table, W, b2d)


_INFO = plsc.get_sparse_core_info()
_NC = _INFO.num_cores
_NS = _INFO.num_subcores
_NW = _NC * _NS

_B = 4096
_L = 50
_NB = _B // _NW
_NBUF = 2


def _make_lookup():
    mesh = plsc.VectorSubcoreMesh(core_axis_name="c", subcore_axis_name="s")

    @functools.partial(
        pl.kernel,
        mesh=mesh,
        out_type=jax.ShapeDtypeStruct(
            (_L, OUT_DIM // 8, _B // 128, 8, 128), jnp.float32),
        scratch_types=(
            [pltpu.VMEM((_NB * _L,), jnp.int32)]
            + [pltpu.VMEM((_NB,), jnp.int32) for _ in range(_NBUF)]
            + [pltpu.VMEM((_NB, OUT_DIM), jnp.float32) for _ in range(_NBUF)]
            + [pltpu.VMEM((OUT_DIM, 129), jnp.float32) for _ in range(_NBUF)]
            + [pltpu.SemaphoreType.DMA for _ in range(2 * _NBUF)]
        ),
        compiler_params=pltpu.CompilerParams(
            use_tc_tiling_on_sc=False, needs_layout_passes=False
        ),
    )
    def lookup_k(idx_hbm, proj_hbm, out_hbm, idx_v, *bufs):
        cols = list(bufs[0:_NBUF])
        rows = list(bufs[_NBUF:2 * _NBUF])
        tiles = list(bufs[2 * _NBUF:3 * _NBUF])
        gsems = list(bufs[3 * _NBUF:4 * _NBUF])
        wsems = list(bufs[4 * _NBUF:5 * _NBUF])
        wid = lax.axis_index("s") * _NC + lax.axis_index("c")
        iota16 = lax.iota(jnp.int32, 16)
        iota50 = iota16 * _L

        pltpu.sync_copy(idx_hbm.at[pl.ds(wid * (_NB * _L), _NB * _L)], idx_v)

        def build_col(l, col):
            for k in range(8):
                vals = plsc.load_gather(idx_v, [iota50 + (16 * _L * k + l)])
                a = ((vals >= _QUARTER).astype(jnp.int32)
                     + (vals >= 2 * _QUARTER).astype(jnp.int32)
                     + (vals >= 3 * _QUARTER).astype(jnp.int32))
                col[pl.ds(16 * k, 16)] = (vals - a * _QUARTER) * 4 + a

        iota16hi = iota16 + 16

        def transpose(rows_v, tile_v):
            for bi in range(_NB):
                v0 = rows_v[bi, pl.ds(0, 16)]
                v1 = rows_v[bi, pl.ds(16, 16)]
                bvec = jnp.zeros((16,), jnp.int32) + bi
                plsc.store_scatter(tile_v, [iota16, bvec], v0)
                plsc.store_scatter(tile_v, [iota16hi, bvec], v1)

        for l in range(_NBUF - 1):
            build_col(l, cols[l])
            pltpu.async_copy(proj_hbm.at[cols[l]], rows[l], gsems[l])

        def block(i, carry):
            for p in range(_NBUF):
                l = i * _NBUF + p
                nxt = (p + _NBUF - 1) % _NBUF

                @pl.when(l + _NBUF - 1 < _L)
                def _():
                    build_col(l + _NBUF - 1, cols[nxt])
                    pltpu.async_copy(proj_hbm.at[cols[nxt]], rows[nxt], gsems[nxt])

                pltpu.make_async_copy(
                    proj_hbm.at[cols[p]], rows[p], gsems[p]
                ).wait()

                @pl.when(i > 0)
                def _():
                    pltpu.make_async_copy(
                        rows[p], proj_hbm.at[pl.ds(0, _NB)], wsems[p]
                    ).wait()

                transpose(rows[p], tiles[p])
                for ot in range(4):
                    pltpu.async_copy(
                        tiles[p].at[pl.ds(ot * 8, 8), pl.ds(0, 128)],
                        out_hbm.at[l, ot, wid],
                        wsems[p],
                    )
            return carry

        lax.fori_loop(0, _L // _NBUF, block, 0, unroll=False)
        for p in range(_NBUF):
            pltpu.make_async_copy(
                rows[p], proj_hbm.at[pl.ds(0, _NB)], wsems[p]
            ).wait()

    return lookup_k


def kernel(x, table, W, b):
    proj = _project_table(table, W, b.reshape(1, OUT_DIM))
    proj = proj.reshape(NUM_EMBEDDINGS, OUT_DIM)
    out6 = _make_lookup()(x.reshape(_B * _L), proj)
    return out6.transpose(2, 4, 0, 1, 3).reshape(_B, _L, OUT_DIM)
